# baseline (speedup 1.0000x reference)
import functools

import jax
import jax.numpy as jnp
from jax import lax
from jax.experimental import pallas as pl
from jax.experimental.pallas import tpu as pltpu
from jax.experimental.pallas import tpu_sc as plsc

N = 10000
E = 320000
D = 128
NC = 2
NS = 16
NW = NC * NS
EPW = E // NW
CH = 80
NCHUNK = EPW // CH
RPT = N // NS

_sc_mesh = plsc.VectorSubcoreMesh(core_axis_name="c", subcore_axis_name="s")


NACC = 10240
RPTD = NACC // NS


@functools.partial(
    pl.kernel,
    out_type=jax.ShapeDtypeStruct((NC, NACC), jnp.float32),
    mesh=_sc_mesh,
    scratch_types=[
        pltpu.VMEM((NCHUNK, 1, CH), jnp.int32),
        pltpu.VMEM((128,), jnp.float32),
        pltpu.VMEM((RPTD,), jnp.float32),
        pltpu.VMEM_SHARED((NACC,), jnp.float32),
    ],
)
def _deg_kernel(dst_hbm, parts_hbm, dstv, ones, zbuf, dacc):
    c = lax.axis_index("c")
    sid = lax.axis_index("s")
    wid = c * NS + sid
    pltpu.sync_copy(dst_hbm.at[wid], dstv)

    def fill_ones(i, _):
        ones[pl.ds(i * 16, 16)] = jnp.full((16,), 1.0, jnp.float32)
        return 0

    lax.fori_loop(0, 128 // 16, fill_ones, 0)

    def fill_zero(i, _):
        zbuf[pl.ds(i * 16, 16)] = jnp.zeros((16,), jnp.float32)
        return 0

    lax.fori_loop(0, RPTD // 16, fill_zero, 0)
    pltpu.sync_copy(zbuf, dacc.at[pl.ds(sid * RPTD, RPTD)])
    plsc.subcore_barrier()

    def count_body(j, _):
        pltpu.sync_copy(ones.at[pl.ds(0, CH)], dacc.at[dstv.at[j, 0]], add=True)
        return 0

    lax.fori_loop(0, NCHUNK, count_body, 0)
    plsc.subcore_barrier()
    rows = pl.ds(sid * RPTD, RPTD)
    pltpu.sync_copy(dacc.at[rows], parts_hbm.at[c, rows])


GRP = 24
NGRP = NCHUNK // GRP
NBUF = 4
EPI = NCHUNK - NGRP * GRP
NACCM = 10112
RPTM = NACCM // NS


@functools.partial(
    pl.kernel,
    out_type=jax.ShapeDtypeStruct((NC, N, D), jnp.float32),
    mesh=_sc_mesh,
    scratch_types=[
        pltpu.VMEM((GRP, 1, CH), jnp.int32),
        pltpu.VMEM((GRP, 1, CH), jnp.int32),
        pltpu.VMEM((CH, D), jnp.float32),
        pltpu.VMEM((CH, D), jnp.float32),
        pltpu.VMEM((CH, D), jnp.float32),
        pltpu.VMEM((CH, D), jnp.float32),
        pltpu.VMEM_SHARED((NACCM, D), jnp.float32),
        pltpu.SemaphoreType.DMA,
        pltpu.SemaphoreType.DMA,
        pltpu.SemaphoreType.DMA,
        pltpu.SemaphoreType.DMA,
        pltpu.SemaphoreType.DMA,
        pltpu.SemaphoreType.DMA,
        pltpu.SemaphoreType.DMA,
        pltpu.SemaphoreType.DMA,
    ],
)
def _msg_kernel(src_hbm, dst_hbm, g_hbm, parts_hbm, si, di, s0, s1, s2, s3,
                acc, gsem0, gsem1, gsem2, gsem3, ssem0, ssem1, ssem2, ssem3):
    c = lax.axis_index("c")
    sid = lax.axis_index("s")
    wid = c * NS + sid
    bufs = (s0, s1, s2, s3)
    gsems = (gsem0, gsem1, gsem2, gsem3)
    ssems = (ssem0, ssem1, ssem2, ssem3)

    def zero_body(i, _):
        r = i // (D // 16)
        q = i % (D // 16)
        s0[r, pl.ds(q * 16, 16)] = jnp.zeros((16,), jnp.float32)
        return 0

    lax.fori_loop(0, CH * (D // 16), zero_body, 0)
    for q in range(7):
        pltpu.sync_copy(s0.at[pl.ds(0, 80)],
                        acc.at[pl.ds(sid * RPTM + q * 80, 80)])
    pltpu.sync_copy(s0.at[pl.ds(0, 72)],
                    acc.at[pl.ds(sid * RPTM + 560, 72)])
    plsc.subcore_barrier()

    def wait_s(b):
        pltpu.make_async_copy(bufs[b], acc.at[di.at[0, 0]], ssems[b]).wait()

    def wait_g(b, k):
        pltpu.make_async_copy(g_hbm.at[si.at[k, 0]], bufs[b],
                              gsems[b]).wait()

    def fire_g(b, k):
        pltpu.async_copy(g_hbm.at[si.at[k, 0]], bufs[b], gsems[b])

    def fire_s(b, k):
        pltpu.async_copy(bufs[b], acc.at[di.at[k, 0]], ssems[b], add=True)

    def load_idx(j0, n):
        pltpu.sync_copy(src_hbm.at[wid, pl.ds(j0, n)], si.at[pl.ds(0, n)])
        pltpu.sync_copy(dst_hbm.at[wid, pl.ds(j0, n)], di.at[pl.ds(0, n)])

    def drain():
        for b in range(NBUF):
            wait_s(b)

    def group_body(t, _):
        @pl.when(t > 0)
        def _():
            drain()

        load_idx(t * GRP, GRP)
        fire_g(0, 0)
        fire_g(1, 1)
        fire_g(2, 2)
        for k in range(GRP):
            b = k % NBUF
            wait_g(b, k)
            fire_s(b, k)
            if k <= GRP - 4:
                if k >= 1:
                    wait_s((k + 3) % NBUF)
                fire_g((k + 3) % NBUF, k + 3)
        return 0

    lax.fori_loop(0, NGRP, group_body, 0)

    drain()
    load_idx(NGRP * GRP, EPI)
    fire_g(0, 0)
    fire_g(1, 1)
    fire_g(2, 2)
    wait_g(0, 0)
    fire_s(0, 0)
    fire_g(3, 3)
    wait_g(1, 1)
    fire_s(1, 1)
    wait_s(0)
    fire_g(0, 4)
    wait_g(2, 2)
    fire_s(2, 2)
    wait_g(3, 3)
    fire_s(3, 3)
    wait_g(0, 4)
    fire_s(0, 4)
    drain()
    plsc.subcore_barrier()

    @pl.when(sid < NS - 1)
    def _():
        rows = pl.ds(sid * RPTM, RPTM)
        pltpu.sync_copy(acc.at[rows], parts_hbm.at[c, rows])

    @pl.when(sid == NS - 1)
    def _():
        rows = pl.ds((NS - 1) * RPTM, N - (NS - 1) * RPTM)
        pltpu.sync_copy(acc.at[rows], parts_hbm.at[c, rows])


def _blk_dinv(pt_ref):
    pt = pt_ref[...]
    return lax.rsqrt(1.0 + pt[:, 0] + pt[:, 1])[:, None]


def _g_body(x_ref, w_ref, pt_ref, g_ref):
    dinv = _blk_dinv(pt_ref)
    h = jnp.dot(x_ref[...], w_ref[...], preferred_element_type=jnp.float32)
    g_ref[...] = dinv * h


def _combine_body(p_ref, g_ref, pt_ref, x_ref, w0_ref, wt_ref, b_ref,
                  out_ref):
    dinv = _blk_dinv(pt_ref)
    b = b_ref[...]
    wt = wt_ref[...]
    s = dinv * (p_ref[0] + p_ref[1] + g_ref[...]) + b
    out_ref[...] = (
        s
        + jnp.dot(s, wt, preferred_element_type=jnp.float32)
        + jnp.dot(x_ref[...], w0_ref[...] - wt,
                  preferred_element_type=jnp.float32)
    )


_BLK = 1000
_GRID = N // _BLK


def _row_blk(i):
    return (i, 0)


def _g_kernel(x, w_conv, dinv):
    return pl.pallas_call(
        _g_body,
        grid=(_GRID,),
        in_specs=[
            pl.BlockSpec((_BLK, D), _row_blk),
            pl.BlockSpec((D, D), lambda i: (0, 0)),
            pl.BlockSpec((_BLK, NC), _row_blk),
        ],
        out_specs=pl.BlockSpec((_BLK, D), _row_blk),
        out_shape=jax.ShapeDtypeStruct((N, D), jnp.float32),
    )(x, w_conv, dinv)


def _combine_kernel(p, g, dinv, x, w0, wt, b2):
    return pl.pallas_call(
        _combine_body,
        grid=(_GRID,),
        in_specs=[
            pl.BlockSpec((NC, _BLK, D), lambda i: (0, i, 0)),
            pl.BlockSpec((_BLK, D), _row_blk),
            pl.BlockSpec((_BLK, NC), _row_blk),
            pl.BlockSpec((_BLK, D), _row_blk),
            pl.BlockSpec((D, D), lambda i: (0, 0)),
            pl.BlockSpec((D, D), lambda i: (0, 0)),
            pl.BlockSpec((1, D), lambda i: (0, 0)),
        ],
        out_specs=pl.BlockSpec((_BLK, D), _row_blk),
        out_shape=jax.ShapeDtypeStruct((N, D), jnp.float32),
    )(p, g, dinv, x, w0, wt, b2)


def kernel(x, edge_index, W_conv, b_conv, W0, Wt):
    src = edge_index[0]
    src_r = src.reshape(NW, NCHUNK, 1, CH)
    dst_r = edge_index[1].reshape(NW, NCHUNK, 1, CH)
    partsT = _deg_kernel(dst_r)[:, :N].T
    g = _g_kernel(x, W_conv, partsT)
    p = _msg_kernel(src_r, dst_r, g)
    return _combine_kernel(p, g, partsT, x, W0, Wt, b_conv.reshape(1, D))


# speedup vs baseline: 1.0332x; 1.0332x over previous
import functools

import jax
import jax.numpy as jnp
from jax import lax
from jax.experimental import pallas as pl
from jax.experimental.pallas import tpu as pltpu
from jax.experimental.pallas import tpu_sc as plsc

N = 10000
E = 320000
D = 128
NC = 2
NS = 16
NW = NC * NS
EPW = E // NW
CH = 100
NCHUNK = EPW // CH
RPT = N // NS

_sc_mesh = plsc.VectorSubcoreMesh(core_axis_name="c", subcore_axis_name="s")


NACC = 10240
RPTD = NACC // NS


@functools.partial(
    pl.kernel,
    out_type=jax.ShapeDtypeStruct((NC, NACC), jnp.float32),
    mesh=_sc_mesh,
    scratch_types=[
        pltpu.VMEM((NCHUNK, 1, CH), jnp.int32),
        pltpu.VMEM((128,), jnp.float32),
        pltpu.VMEM((RPTD,), jnp.float32),
        pltpu.VMEM_SHARED((NACC,), jnp.float32),
    ],
)
def _deg_kernel(dst_hbm, parts_hbm, dstv, ones, zbuf, dacc):
    c = lax.axis_index("c")
    sid = lax.axis_index("s")
    wid = c * NS + sid
    pltpu.sync_copy(dst_hbm.at[wid], dstv)

    def fill_ones(i, _):
        ones[pl.ds(i * 16, 16)] = jnp.full((16,), 1.0, jnp.float32)
        return 0

    lax.fori_loop(0, 128 // 16, fill_ones, 0)

    def fill_zero(i, _):
        zbuf[pl.ds(i * 16, 16)] = jnp.zeros((16,), jnp.float32)
        return 0

    lax.fori_loop(0, RPTD // 16, fill_zero, 0)
    pltpu.sync_copy(zbuf, dacc.at[pl.ds(sid * RPTD, RPTD)])
    plsc.subcore_barrier()

    def count_body(j, _):
        pltpu.sync_copy(ones.at[pl.ds(0, CH)], dacc.at[dstv.at[j, 0]], add=True)
        return 0

    lax.fori_loop(0, NCHUNK, count_body, 0)
    plsc.subcore_barrier()
    rows = pl.ds(sid * RPTD, RPTD)
    pltpu.sync_copy(dacc.at[rows], parts_hbm.at[c, rows])


GRP = 24
NGRP = NCHUNK // GRP
NBUF = 3
EPI = NCHUNK - NGRP * GRP
NACCM = 10112
RPTM = NACCM // NS


@functools.partial(
    pl.kernel,
    out_type=jax.ShapeDtypeStruct((NC, N, D), jnp.float32),
    mesh=_sc_mesh,
    scratch_types=[
        pltpu.VMEM((GRP, 1, CH), jnp.int32),
        pltpu.VMEM((GRP, 1, CH), jnp.int32),
        pltpu.VMEM((CH, D), jnp.float32),
        pltpu.VMEM((CH, D), jnp.float32),
        pltpu.VMEM((CH, D), jnp.float32),
        pltpu.VMEM_SHARED((NACCM, D), jnp.float32),
        pltpu.SemaphoreType.DMA,
        pltpu.SemaphoreType.DMA,
        pltpu.SemaphoreType.DMA,
        pltpu.SemaphoreType.DMA,
        pltpu.SemaphoreType.DMA,
        pltpu.SemaphoreType.DMA,
    ],
)
def _msg_kernel(src_hbm, dst_hbm, g_hbm, parts_hbm, si, di, s0, s1, s2,
                acc, gsem0, gsem1, gsem2, ssem0, ssem1, ssem2):
    c = lax.axis_index("c")
    sid = lax.axis_index("s")
    wid = c * NS + sid
    bufs = (s0, s1, s2)
    gsems = (gsem0, gsem1, gsem2)
    ssems = (ssem0, ssem1, ssem2)

    def zero_body(i, _):
        r = i // (D // 16)
        q = i % (D // 16)
        s0[r, pl.ds(q * 16, 16)] = jnp.zeros((16,), jnp.float32)
        return 0

    lax.fori_loop(0, CH * (D // 16), zero_body, 0)
    for q in range(6):
        pltpu.sync_copy(s0.at[pl.ds(0, 96)],
                        acc.at[pl.ds(sid * RPTM + q * 96, 96)])
    pltpu.sync_copy(s0.at[pl.ds(0, 56)],
                    acc.at[pl.ds(sid * RPTM + 576, 56)])
    plsc.subcore_barrier()

    def wait_s(b):
        pltpu.make_async_copy(bufs[b], acc.at[di.at[0, 0]], ssems[b]).wait()

    def wait_g(b, k):
        pltpu.make_async_copy(g_hbm.at[si.at[k, 0]], bufs[b],
                              gsems[b]).wait()

    def fire_g(b, k):
        pltpu.async_copy(g_hbm.at[si.at[k, 0]], bufs[b], gsems[b])

    def fire_s(b, k):
        pltpu.async_copy(bufs[b], acc.at[di.at[k, 0]], ssems[b], add=True)

    def load_idx(j0, n):
        pltpu.sync_copy(src_hbm.at[wid, pl.ds(j0, n)], si.at[pl.ds(0, n)])
        pltpu.sync_copy(dst_hbm.at[wid, pl.ds(j0, n)], di.at[pl.ds(0, n)])

    def drain3():
        for b in range(NBUF):
            wait_s(b)

    def group_body(t, _):
        @pl.when(t > 0)
        def _():
            drain3()

        load_idx(t * GRP, GRP)
        fire_g(0, 0)
        fire_g(1, 1)
        for k in range(GRP):
            b = k % NBUF
            wait_g(b, k)
            fire_s(b, k)
            if k <= GRP - 3:
                if k >= 1:
                    wait_s((k + 2) % NBUF)
                fire_g((k + 2) % NBUF, k + 2)
        return 0

    lax.fori_loop(0, NGRP, group_body, 0)

    drain3()
    load_idx(NGRP * GRP, EPI)
    fire_g(0, 0)
    fire_g(1, 1)
    wait_g(0, 0)
    fire_s(0, 0)
    fire_g(2, 2)
    wait_g(1, 1)
    fire_s(1, 1)
    wait_s(0)
    fire_g(0, 3)
    wait_g(2, 2)
    fire_s(2, 2)
    wait_g(0, 3)
    fire_s(0, 3)
    drain3()
    plsc.subcore_barrier()

    @pl.when(sid < NS - 1)
    def _():
        rows = pl.ds(sid * RPTM, RPTM)
        pltpu.sync_copy(acc.at[rows], parts_hbm.at[c, rows])

    @pl.when(sid == NS - 1)
    def _():
        rows = pl.ds((NS - 1) * RPTM, N - (NS - 1) * RPTM)
        pltpu.sync_copy(acc.at[rows], parts_hbm.at[c, rows])


def _blk_dinv(pt_ref):
    pt = pt_ref[...]
    return lax.rsqrt(1.0 + pt[:, 0] + pt[:, 1])[:, None]


def _g_body(x_ref, w_ref, pt_ref, g_ref):
    dinv = _blk_dinv(pt_ref)
    h = jnp.dot(x_ref[...], w_ref[...], preferred_element_type=jnp.float32)
    g_ref[...] = dinv * h


def _combine_body(p_ref, g_ref, pt_ref, x_ref, w0_ref, wt_ref, b_ref,
                  out_ref):
    dinv = _blk_dinv(pt_ref)
    b = b_ref[...]
    wt = wt_ref[...]
    s = dinv * (p_ref[0] + p_ref[1] + g_ref[...]) + b
    out_ref[...] = (
        s
        + jnp.dot(s, wt, preferred_element_type=jnp.float32)
        + jnp.dot(x_ref[...], w0_ref[...] - wt,
                  preferred_element_type=jnp.float32)
    )


_BLK = 1000
_GRID = N // _BLK


def _row_blk(i):
    return (i, 0)


def _g_kernel(x, w_conv, dinv):
    return pl.pallas_call(
        _g_body,
        grid=(_GRID,),
        in_specs=[
            pl.BlockSpec((_BLK, D), _row_blk),
            pl.BlockSpec((D, D), lambda i: (0, 0)),
            pl.BlockSpec((_BLK, NC), _row_blk),
        ],
        out_specs=pl.BlockSpec((_BLK, D), _row_blk),
        out_shape=jax.ShapeDtypeStruct((N, D), jnp.float32),
    )(x, w_conv, dinv)


def _combine_kernel(p, g, dinv, x, w0, wt, b2):
    return pl.pallas_call(
        _combine_body,
        grid=(_GRID,),
        in_specs=[
            pl.BlockSpec((NC, _BLK, D), lambda i: (0, i, 0)),
            pl.BlockSpec((_BLK, D), _row_blk),
            pl.BlockSpec((_BLK, NC), _row_blk),
            pl.BlockSpec((_BLK, D), _row_blk),
            pl.BlockSpec((D, D), lambda i: (0, 0)),
            pl.BlockSpec((D, D), lambda i: (0, 0)),
            pl.BlockSpec((1, D), lambda i: (0, 0)),
        ],
        out_specs=pl.BlockSpec((_BLK, D), _row_blk),
        out_shape=jax.ShapeDtypeStruct((N, D), jnp.float32),
    )(p, g, dinv, x, w0, wt, b2)


def kernel(x, edge_index, W_conv, b_conv, W0, Wt):
    src = edge_index[0]
    src_r = src.reshape(NW, NCHUNK, 1, CH)
    dst_r = edge_index[1].reshape(NW, NCHUNK, 1, CH)
    partsT = _deg_kernel(dst_r)[:, :N].T
    g = _g_kernel(x, W_conv, partsT)
    p = _msg_kernel(src_r, dst_r, g)
    return _combine_kernel(p, g, partsT, x, W0, Wt, b_conv.reshape(1, D))


# speedup vs baseline: 1.0565x; 1.0226x over previous
import functools

import jax
import jax.numpy as jnp
from jax import lax
from jax.experimental import pallas as pl
from jax.experimental.pallas import tpu as pltpu
from jax.experimental.pallas import tpu_sc as plsc

N = 10000
E = 320000
D = 128
NC = 2
NS = 16
NW = NC * NS
EPW = E // NW
CH = 100
NCHUNK = EPW // CH
RPT = N // NS

_sc_mesh = plsc.VectorSubcoreMesh(core_axis_name="c", subcore_axis_name="s")


NACC = 10240
RPTD = NACC // NS


@functools.partial(
    pl.kernel,
    out_type=jax.ShapeDtypeStruct((NC, NACC), jnp.float32),
    mesh=_sc_mesh,
    scratch_types=[
        pltpu.VMEM((NCHUNK, 1, CH), jnp.int32),
        pltpu.VMEM((128,), jnp.float32),
        pltpu.VMEM((RPTD,), jnp.float32),
        pltpu.VMEM_SHARED((NACC,), jnp.float32),
        pltpu.SemaphoreType.DMA,
        pltpu.SemaphoreType.DMA,
    ],
)
def _deg_kernel(dst_hbm, parts_hbm, dstv, ones, zbuf, dacc, csem0, csem1):
    c = lax.axis_index("c")
    sid = lax.axis_index("s")
    wid = c * NS + sid
    pltpu.sync_copy(dst_hbm.at[wid], dstv)

    def fill_ones(i, _):
        ones[pl.ds(i * 16, 16)] = jnp.full((16,), 1.0, jnp.float32)
        return 0

    lax.fori_loop(0, 128 // 16, fill_ones, 0)

    def fill_zero(i, _):
        zbuf[pl.ds(i * 16, 16)] = jnp.zeros((16,), jnp.float32)
        return 0

    lax.fori_loop(0, RPTD // 16, fill_zero, 0)
    pltpu.sync_copy(zbuf, dacc.at[pl.ds(sid * RPTD, RPTD)])
    plsc.subcore_barrier()

    csems = (csem0, csem1)

    def fire_c(j, b):
        pltpu.async_copy(ones.at[pl.ds(0, CH)], dacc.at[dstv.at[j, 0]],
                         csems[b], add=True)

    def wait_c(b):
        pltpu.make_async_copy(ones.at[pl.ds(0, CH)], dacc.at[dstv.at[0, 0]],
                              csems[b]).wait()

    def count_body(t, _):
        j = 2 * t
        fire_c(j, 0)
        fire_c(j + 1, 1)
        wait_c(0)
        wait_c(1)
        return 0

    lax.fori_loop(0, NCHUNK // 2, count_body, 0)
    plsc.subcore_barrier()
    rows = pl.ds(sid * RPTD, RPTD)
    pltpu.sync_copy(dacc.at[rows], parts_hbm.at[c, rows])


GRP = 24
NGRP = NCHUNK // GRP
NBUF = 3
EPI = NCHUNK - NGRP * GRP
NACCM = 10112
RPTM = NACCM // NS


@functools.partial(
    pl.kernel,
    out_type=jax.ShapeDtypeStruct((NC, N, D), jnp.float32),
    mesh=_sc_mesh,
    scratch_types=[
        pltpu.VMEM((GRP, 1, CH), jnp.int32),
        pltpu.VMEM((GRP, 1, CH), jnp.int32),
        pltpu.VMEM((CH, D), jnp.float32),
        pltpu.VMEM((CH, D), jnp.float32),
        pltpu.VMEM((CH, D), jnp.float32),
        pltpu.VMEM_SHARED((NACCM, D), jnp.float32),
        pltpu.SemaphoreType.DMA,
        pltpu.SemaphoreType.DMA,
        pltpu.SemaphoreType.DMA,
        pltpu.SemaphoreType.DMA,
        pltpu.SemaphoreType.DMA,
        pltpu.SemaphoreType.DMA,
    ],
)
def _msg_kernel(src_hbm, dst_hbm, g_hbm, parts_hbm, si, di, s0, s1, s2,
                acc, gsem0, gsem1, gsem2, ssem0, ssem1, ssem2):
    c = lax.axis_index("c")
    sid = lax.axis_index("s")
    wid = c * NS + sid
    bufs = (s0, s1, s2)
    gsems = (gsem0, gsem1, gsem2)
    ssems = (ssem0, ssem1, ssem2)

    def zero_body(i, _):
        r = i // (D // 16)
        q = i % (D // 16)
        s0[r, pl.ds(q * 16, 16)] = jnp.zeros((16,), jnp.float32)
        return 0

    lax.fori_loop(0, CH * (D // 16), zero_body, 0)
    for q in range(6):
        pltpu.sync_copy(s0.at[pl.ds(0, 96)],
                        acc.at[pl.ds(sid * RPTM + q * 96, 96)])
    pltpu.sync_copy(s0.at[pl.ds(0, 56)],
                    acc.at[pl.ds(sid * RPTM + 576, 56)])
    plsc.subcore_barrier()

    def wait_s(b):
        pltpu.make_async_copy(bufs[b], acc.at[di.at[0, 0]], ssems[b]).wait()

    def wait_g(b, k):
        pltpu.make_async_copy(g_hbm.at[si.at[k, 0]], bufs[b],
                              gsems[b]).wait()

    def fire_g(b, k):
        pltpu.async_copy(g_hbm.at[si.at[k, 0]], bufs[b], gsems[b])

    def fire_s(b, k):
        pltpu.async_copy(bufs[b], acc.at[di.at[k, 0]], ssems[b], add=True)

    def load_idx(j0, n):
        pltpu.sync_copy(src_hbm.at[wid, pl.ds(j0, n)], si.at[pl.ds(0, n)])
        pltpu.sync_copy(dst_hbm.at[wid, pl.ds(j0, n)], di.at[pl.ds(0, n)])

    def drain3():
        for b in range(NBUF):
            wait_s(b)

    def group_body(t, _):
        @pl.when(t > 0)
        def _():
            drain3()

        load_idx(t * GRP, GRP)
        fire_g(0, 0)
        fire_g(1, 1)
        for k in range(GRP):
            b = k % NBUF
            wait_g(b, k)
            fire_s(b, k)
            if k <= GRP - 3:
                if k >= 1:
                    wait_s((k + 2) % NBUF)
                fire_g((k + 2) % NBUF, k + 2)
        return 0

    lax.fori_loop(0, NGRP, group_body, 0)

    drain3()
    load_idx(NGRP * GRP, EPI)
    fire_g(0, 0)
    fire_g(1, 1)
    wait_g(0, 0)
    fire_s(0, 0)
    fire_g(2, 2)
    wait_g(1, 1)
    fire_s(1, 1)
    wait_s(0)
    fire_g(0, 3)
    wait_g(2, 2)
    fire_s(2, 2)
    wait_g(0, 3)
    fire_s(0, 3)
    drain3()
    plsc.subcore_barrier()

    @pl.when(sid < NS - 1)
    def _():
        rows = pl.ds(sid * RPTM, RPTM)
        pltpu.sync_copy(acc.at[rows], parts_hbm.at[c, rows])

    @pl.when(sid == NS - 1)
    def _():
        rows = pl.ds((NS - 1) * RPTM, N - (NS - 1) * RPTM)
        pltpu.sync_copy(acc.at[rows], parts_hbm.at[c, rows])


def _blk_dinv(pt_ref):
    pt = pt_ref[...]
    return lax.rsqrt(1.0 + pt[:, 0] + pt[:, 1])[:, None]


def _g_body(x_ref, w_ref, pt_ref, g_ref):
    dinv = _blk_dinv(pt_ref)
    h = jnp.dot(x_ref[...], w_ref[...], preferred_element_type=jnp.float32)
    g_ref[...] = dinv * h


def _combine_body(p_ref, g_ref, pt_ref, x_ref, w0_ref, wt_ref, b_ref,
                  out_ref):
    dinv = _blk_dinv(pt_ref)
    b = b_ref[...]
    wt = wt_ref[...]
    s = dinv * (p_ref[0] + p_ref[1] + g_ref[...]) + b
    out_ref[...] = (
        s
        + jnp.dot(s, wt, preferred_element_type=jnp.float32)
        + jnp.dot(x_ref[...], w0_ref[...] - wt,
                  preferred_element_type=jnp.float32)
    )


_BLK = 1000
_GRID = N // _BLK


def _row_blk(i):
    return (i, 0)


def _g_kernel(x, w_conv, dinv):
    return pl.pallas_call(
        _g_body,
        grid=(_GRID,),
        in_specs=[
            pl.BlockSpec((_BLK, D), _row_blk),
            pl.BlockSpec((D, D), lambda i: (0, 0)),
            pl.BlockSpec((_BLK, NC), _row_blk),
        ],
        out_specs=pl.BlockSpec((_BLK, D), _row_blk),
        out_shape=jax.ShapeDtypeStruct((N, D), jnp.float32),
    )(x, w_conv, dinv)


def _combine_kernel(p, g, dinv, x, w0, wt, b2):
    return pl.pallas_call(
        _combine_body,
        grid=(_GRID,),
        in_specs=[
            pl.BlockSpec((NC, _BLK, D), lambda i: (0, i, 0)),
            pl.BlockSpec((_BLK, D), _row_blk),
            pl.BlockSpec((_BLK, NC), _row_blk),
            pl.BlockSpec((_BLK, D), _row_blk),
            pl.BlockSpec((D, D), lambda i: (0, 0)),
            pl.BlockSpec((D, D), lambda i: (0, 0)),
            pl.BlockSpec((1, D), lambda i: (0, 0)),
        ],
        out_specs=pl.BlockSpec((_BLK, D), _row_blk),
        out_shape=jax.ShapeDtypeStruct((N, D), jnp.float32),
    )(p, g, dinv, x, w0, wt, b2)


def kernel(x, edge_index, W_conv, b_conv, W0, Wt):
    src = edge_index[0]
    src_r = src.reshape(NW, NCHUNK, 1, CH)
    dst_r = edge_index[1].reshape(NW, NCHUNK, 1, CH)
    partsT = _deg_kernel(dst_r)[:, :N].T
    g = _g_kernel(x, W_conv, partsT)
    p = _msg_kernel(src_r, dst_r, g)
    return _combine_kernel(p, g, partsT, x, W0, Wt, b_conv.reshape(1, D))


# speedup vs baseline: 1.0620x; 1.0051x over previous
import functools

import jax
import jax.numpy as jnp
from jax import lax
from jax.experimental import pallas as pl
from jax.experimental.pallas import tpu as pltpu
from jax.experimental.pallas import tpu_sc as plsc

N = 10000
E = 320000
D = 128
NC = 2
NS = 16
NW = NC * NS
EPW = E // NW
CH = 100
NCHUNK = EPW // CH
RPT = N // NS

_sc_mesh = plsc.VectorSubcoreMesh(core_axis_name="c", subcore_axis_name="s")


NACC = 10240
RPTD = NACC // NS


@functools.partial(
    pl.kernel,
    out_type=jax.ShapeDtypeStruct((NC, NACC), jnp.float32),
    mesh=_sc_mesh,
    scratch_types=[
        pltpu.VMEM((NCHUNK, 1, CH), jnp.int32),
        pltpu.VMEM((128,), jnp.float32),
        pltpu.VMEM((RPTD,), jnp.float32),
        pltpu.VMEM_SHARED((NACC,), jnp.float32),
        pltpu.SemaphoreType.DMA,
        pltpu.SemaphoreType.DMA,
        pltpu.SemaphoreType.DMA,
        pltpu.SemaphoreType.DMA,
    ],
)
def _deg_kernel(dst_hbm, parts_hbm, dstv, ones, zbuf, dacc, csem0, csem1,
                csem2, csem3):
    c = lax.axis_index("c")
    sid = lax.axis_index("s")
    wid = c * NS + sid
    pltpu.sync_copy(dst_hbm.at[wid], dstv)

    def fill_ones(i, _):
        ones[pl.ds(i * 16, 16)] = jnp.full((16,), 1.0, jnp.float32)
        return 0

    lax.fori_loop(0, 128 // 16, fill_ones, 0)

    def fill_zero(i, _):
        zbuf[pl.ds(i * 16, 16)] = jnp.zeros((16,), jnp.float32)
        return 0

    lax.fori_loop(0, RPTD // 16, fill_zero, 0)
    pltpu.sync_copy(zbuf, dacc.at[pl.ds(sid * RPTD, RPTD)])
    plsc.subcore_barrier()

    csems = (csem0, csem1, csem2, csem3)

    def fire_c(j, b):
        pltpu.async_copy(ones.at[pl.ds(0, CH)], dacc.at[dstv.at[j, 0]],
                         csems[b], add=True)

    def wait_c(b):
        pltpu.make_async_copy(ones.at[pl.ds(0, CH)], dacc.at[dstv.at[0, 0]],
                              csems[b]).wait()

    def count_body(t, _):
        j = 4 * t
        for b in range(4):
            fire_c(j + b, b)
        for b in range(4):
            wait_c(b)
        return 0

    lax.fori_loop(0, NCHUNK // 4, count_body, 0)
    plsc.subcore_barrier()
    rows = pl.ds(sid * RPTD, RPTD)
    pltpu.sync_copy(dacc.at[rows], parts_hbm.at[c, rows])


GRP = 24
NGRP = NCHUNK // GRP
NBUF = 3
EPI = NCHUNK - NGRP * GRP
NACCM = 10112
RPTM = NACCM // NS


@functools.partial(
    pl.kernel,
    out_type=jax.ShapeDtypeStruct((NC, N, D), jnp.float32),
    mesh=_sc_mesh,
    scratch_types=[
        pltpu.VMEM((GRP, 1, CH), jnp.int32),
        pltpu.VMEM((GRP, 1, CH), jnp.int32),
        pltpu.VMEM((CH, D), jnp.float32),
        pltpu.VMEM((CH, D), jnp.float32),
        pltpu.VMEM((CH, D), jnp.float32),
        pltpu.VMEM_SHARED((NACCM, D), jnp.float32),
        pltpu.SemaphoreType.DMA,
        pltpu.SemaphoreType.DMA,
        pltpu.SemaphoreType.DMA,
        pltpu.SemaphoreType.DMA,
        pltpu.SemaphoreType.DMA,
        pltpu.SemaphoreType.DMA,
    ],
)
def _msg_kernel(src_hbm, dst_hbm, g_hbm, parts_hbm, si, di, s0, s1, s2,
                acc, gsem0, gsem1, gsem2, ssem0, ssem1, ssem2):
    c = lax.axis_index("c")
    sid = lax.axis_index("s")
    wid = c * NS + sid
    bufs = (s0, s1, s2)
    gsems = (gsem0, gsem1, gsem2)
    ssems = (ssem0, ssem1, ssem2)

    def zero_body(i, _):
        r = i // (D // 16)
        q = i % (D // 16)
        s0[r, pl.ds(q * 16, 16)] = jnp.zeros((16,), jnp.float32)
        return 0

    lax.fori_loop(0, CH * (D // 16), zero_body, 0)
    for q in range(6):
        pltpu.sync_copy(s0.at[pl.ds(0, 96)],
                        acc.at[pl.ds(sid * RPTM + q * 96, 96)])
    pltpu.sync_copy(s0.at[pl.ds(0, 56)],
                    acc.at[pl.ds(sid * RPTM + 576, 56)])
    plsc.subcore_barrier()

    def wait_s(b):
        pltpu.make_async_copy(bufs[b], acc.at[di.at[0, 0]], ssems[b]).wait()

    def wait_g(b, k):
        pltpu.make_async_copy(g_hbm.at[si.at[k, 0]], bufs[b],
                              gsems[b]).wait()

    def fire_g(b, k):
        pltpu.async_copy(g_hbm.at[si.at[k, 0]], bufs[b], gsems[b])

    def fire_s(b, k):
        pltpu.async_copy(bufs[b], acc.at[di.at[k, 0]], ssems[b], add=True)

    def load_idx(j0, n):
        pltpu.sync_copy(src_hbm.at[wid, pl.ds(j0, n)], si.at[pl.ds(0, n)])
        pltpu.sync_copy(dst_hbm.at[wid, pl.ds(j0, n)], di.at[pl.ds(0, n)])

    def drain3():
        for b in range(NBUF):
            wait_s(b)

    def group_body(t, _):
        @pl.when(t > 0)
        def _():
            drain3()

        load_idx(t * GRP, GRP)
        fire_g(0, 0)
        fire_g(1, 1)
        for k in range(GRP):
            b = k % NBUF
            wait_g(b, k)
            fire_s(b, k)
            if k <= GRP - 3:
                if k >= 1:
                    wait_s((k + 2) % NBUF)
                fire_g((k + 2) % NBUF, k + 2)
        return 0

    lax.fori_loop(0, NGRP, group_body, 0)

    drain3()
    load_idx(NGRP * GRP, EPI)
    fire_g(0, 0)
    fire_g(1, 1)
    wait_g(0, 0)
    fire_s(0, 0)
    fire_g(2, 2)
    wait_g(1, 1)
    fire_s(1, 1)
    wait_s(0)
    fire_g(0, 3)
    wait_g(2, 2)
    fire_s(2, 2)
    wait_g(0, 3)
    fire_s(0, 3)
    drain3()
    plsc.subcore_barrier()

    @pl.when(sid < NS - 1)
    def _():
        rows = pl.ds(sid * RPTM, RPTM)
        pltpu.sync_copy(acc.at[rows], parts_hbm.at[c, rows])

    @pl.when(sid == NS - 1)
    def _():
        rows = pl.ds((NS - 1) * RPTM, N - (NS - 1) * RPTM)
        pltpu.sync_copy(acc.at[rows], parts_hbm.at[c, rows])


def _blk_dinv(pt_ref):
    pt = pt_ref[...]
    return lax.rsqrt(1.0 + pt[:, 0] + pt[:, 1])[:, None]


def _g_body(x_ref, w_ref, pt_ref, g_ref):
    dinv = _blk_dinv(pt_ref)
    h = jnp.dot(x_ref[...], w_ref[...], preferred_element_type=jnp.float32)
    g_ref[...] = dinv * h


def _combine_body(p_ref, g_ref, pt_ref, x_ref, w0_ref, wt_ref, b_ref,
                  out_ref):
    dinv = _blk_dinv(pt_ref)
    b = b_ref[...]
    wt = wt_ref[...]
    s = dinv * (p_ref[0] + p_ref[1] + g_ref[...]) + b
    out_ref[...] = (
        s
        + jnp.dot(s, wt, preferred_element_type=jnp.float32)
        + jnp.dot(x_ref[...], w0_ref[...] - wt,
                  preferred_element_type=jnp.float32)
    )


_BLK = 1000
_GRID = N // _BLK


def _row_blk(i):
    return (i, 0)


def _g_kernel(x, w_conv, dinv):
    return pl.pallas_call(
        _g_body,
        grid=(_GRID,),
        in_specs=[
            pl.BlockSpec((_BLK, D), _row_blk),
            pl.BlockSpec((D, D), lambda i: (0, 0)),
            pl.BlockSpec((_BLK, NC), _row_blk),
        ],
        out_specs=pl.BlockSpec((_BLK, D), _row_blk),
        out_shape=jax.ShapeDtypeStruct((N, D), jnp.float32),
    )(x, w_conv, dinv)


def _combine_kernel(p, g, dinv, x, w0, wt, b2):
    return pl.pallas_call(
        _combine_body,
        grid=(_GRID,),
        in_specs=[
            pl.BlockSpec((NC, _BLK, D), lambda i: (0, i, 0)),
            pl.BlockSpec((_BLK, D), _row_blk),
            pl.BlockSpec((_BLK, NC), _row_blk),
            pl.BlockSpec((_BLK, D), _row_blk),
            pl.BlockSpec((D, D), lambda i: (0, 0)),
            pl.BlockSpec((D, D), lambda i: (0, 0)),
            pl.BlockSpec((1, D), lambda i: (0, 0)),
        ],
        out_specs=pl.BlockSpec((_BLK, D), _row_blk),
        out_shape=jax.ShapeDtypeStruct((N, D), jnp.float32),
    )(p, g, dinv, x, w0, wt, b2)


def kernel(x, edge_index, W_conv, b_conv, W0, Wt):
    src = edge_index[0]
    src_r = src.reshape(NW, NCHUNK, 1, CH)
    dst_r = edge_index[1].reshape(NW, NCHUNK, 1, CH)
    partsT = _deg_kernel(dst_r)[:, :N].T
    g = _g_kernel(x, W_conv, partsT)
    p = _msg_kernel(src_r, dst_r, g)
    return _combine_kernel(p, g, partsT, x, W0, Wt, b_conv.reshape(1, D))
